# baseline (speedup 1.0000x reference)
import jax
import jax.numpy as jnp
from jax import lax
from jax.experimental import pallas as pl
from jax.experimental.pallas import tpu as pltpu, tpu_sc as plsc
import functools

_T_SKIP = 4
_PRED_STEPS = 12
_NUM_NEG = 10
_B, _T, _D = 4, 512, 128
_TARGET_LEN = _T - _T_SKIP - _PRED_STEPS - 1
_NSLAB = (_NUM_NEG + 1) * _TARGET_LEN * _PRED_STEPS
_N = _NSLAB * _B

_NC, _NS = 2, 16
_NW = _NC * _NS
_R = 8160
_TAIL = _N - _NW * _R
_C = 408
_NCHUNK = _R // _C
_NBUF = 2
_NPAIR = _NCHUNK // _NBUF


def _gather_body(x_hbm, gidx_hbm, out_hbm, idx_v, tidx_v, buf0, buf1, g0, g1, s0, s1):
    bufs = (buf0, buf1)
    gsem = (g0, g1)
    ssem = (s0, s1)
    wid = lax.axis_index("s") * _NC + lax.axis_index("c")
    base = wid * _R
    pltpu.sync_copy(gidx_hbm.at[pl.ds(base, _R)], idx_v)

    def pair(t, _):
        for b in range(_NBUF):
            @pl.when(t > 0)
            def _():
                pltpu.make_async_copy(
                    bufs[b], out_hbm.at[pl.ds(0, _C)], ssem[b]
                ).wait()

            off = (t * _NBUF + b) * _C
            pltpu.async_copy(
                x_hbm.at[idx_v.at[pl.ds(off, _C)]], bufs[b], gsem[b]
            ).wait()
            pltpu.async_copy(bufs[b], out_hbm.at[pl.ds(base + off, _C)], ssem[b])
        return ()

    lax.fori_loop(0, _NPAIR, pair, (), unroll=False)
    for b in range(_NBUF):
        pltpu.make_async_copy(bufs[b], out_hbm.at[pl.ds(0, _C)], ssem[b]).wait()

    @pl.when(wid == _NW - 1)
    def _():
        tbase = _NW * _R
        pltpu.sync_copy(gidx_hbm.at[pl.ds(tbase, _TAIL)], tidx_v)
        pltpu.async_copy(
            x_hbm.at[tidx_v], bufs[0].at[pl.ds(0, _TAIL)], gsem[0]
        ).wait()
        pltpu.async_copy(
            bufs[0].at[pl.ds(0, _TAIL)], out_hbm.at[pl.ds(tbase, _TAIL)], ssem[0]
        ).wait()


@functools.partial(jax.jit)
def _sc_gather(x_tb, g_idx):
    mesh = plsc.VectorSubcoreMesh(
        core_axis_name="c", subcore_axis_name="s", num_cores=_NC, num_subcores=_NS
    )
    return pl.kernel(
        _gather_body,
        out_type=jax.ShapeDtypeStruct((_N, _D), jnp.float32),
        mesh=mesh,
        scratch_types=(
            [pltpu.VMEM((_R,), jnp.int32), pltpu.VMEM((_TAIL,), jnp.int32)]
            + [pltpu.VMEM((_C, _D), jnp.float32)] * _NBUF
            + [pltpu.SemaphoreType.DMA] * (2 * _NBUF)
        ),
    )(x_tb, g_idx)


def kernel(inputs, neg_indices):
    x_flat = inputs.reshape(_B * _T, _D)
    l = jnp.arange(_TARGET_LEN, dtype=jnp.int32)
    p = jnp.arange(_PRED_STEPS, dtype=jnp.int32)
    pos = (_T_SKIP + 1 + l[:, None] + p[None, :]).reshape(-1)
    full = jnp.concatenate([pos, neg_indices])
    g_idx = (
        full[:, None] + _T * jnp.arange(_B, dtype=jnp.int32)[None, :]
    ).reshape(-1)
    out = _sc_gather(x_flat, g_idx)
    out = out.reshape(_NUM_NEG + 1, _TARGET_LEN, _PRED_STEPS, _B, _D)
    return jnp.transpose(out, (3, 0, 1, 2, 4))


# speedup vs baseline: 1.4286x; 1.4286x over previous
import jax
import jax.numpy as jnp
from jax import lax
from jax.experimental import pallas as pl
from jax.experimental.pallas import tpu as pltpu, tpu_sc as plsc
import functools

_T_SKIP = 4
_PRED_STEPS = 12
_NUM_NEG = 10
_B, _T, _D = 4, 512, 128
_TARGET_LEN = _T - _T_SKIP - _PRED_STEPS - 1
_NSLAB = (_NUM_NEG + 1) * _TARGET_LEN * _PRED_STEPS
_N = _NSLAB * _B

_NC, _NS = 2, 16
_NW = _NC * _NS
_RS = 2040
_R = _RS * _B
_TS = _NSLAB - _NW * _RS
_TPAD = 64
_C = 480
_CS = _C // _B
_NCHUNK = _R // _C
_NPAIR = (_NCHUNK - 1) // 2

_LANE = 16


def _build_idx(idx_slab, idx_c, nrows, row0):
    iota = lax.iota(jnp.int32, _LANE)
    for k in range(nrows // _LANE):
        r = row0 + k * _LANE + iota
        t = plsc.load_gather(idx_slab, [lax.shift_right_logical(r, 2)])
        idx_c[pl.ds(k * _LANE, _LANE)] = t * _B + lax.bitwise_and(r, 3)


def _gather_body(x_hbm, slab_hbm, out_hbm, idx_t, idx_tt, ic0, ic1,
                 buf0, buf1, g0, g1, s0, s1):
    ics = (ic0, ic1)
    bufs = (buf0, buf1)
    gsem = (g0, g1)
    ssem = (s0, s1)
    wid = lax.axis_index("s") * _NC + lax.axis_index("c")
    base = wid * _R
    pltpu.sync_copy(slab_hbm.at[pl.ds(wid * _RS, _RS)], idx_t)

    def chunk(c, p, drain):
        if drain:
            pltpu.make_async_copy(bufs[p], out_hbm.at[pl.ds(0, _C)], ssem[p]).wait()
        _build_idx(idx_t, ics[p], _C, c * _C)
        pltpu.async_copy(x_hbm.at[ics[p]], bufs[p], gsem[p]).wait()
        pltpu.async_copy(bufs[p], out_hbm.at[pl.ds(base + c * _C, _C)], ssem[p])

    chunk(0, 0, False)

    def pair(t, _):
        @pl.when(t > 0)
        def _():
            pltpu.make_async_copy(bufs[1], out_hbm.at[pl.ds(0, _C)], ssem[1]).wait()

        _build_idx(idx_t, ics[1], _C, (2 * t + 1) * _C)
        pltpu.async_copy(x_hbm.at[ics[1]], bufs[1], gsem[1]).wait()
        pltpu.async_copy(
            bufs[1], out_hbm.at[pl.ds(base + (2 * t + 1) * _C, _C)], ssem[1]
        )
        chunk(2 * t + 2, 0, True)
        return ()

    lax.fori_loop(0, _NPAIR, pair, (), unroll=False)
    for p in range(2):
        pltpu.make_async_copy(bufs[p], out_hbm.at[pl.ds(0, _C)], ssem[p]).wait()

    @pl.when(wid == _NW - 1)
    def _():
        nrows = _TS * _B
        pltpu.sync_copy(slab_hbm.at[pl.ds(_NW * _RS, _TPAD)], idx_tt)
        _build_idx(idx_tt, ic0, nrows, 0)
        pltpu.async_copy(
            x_hbm.at[ic0.at[pl.ds(0, nrows)]], buf0.at[pl.ds(0, nrows)], g0
        ).wait()
        pltpu.async_copy(
            buf0.at[pl.ds(0, nrows)], out_hbm.at[pl.ds(_NW * _R, nrows)], s0
        ).wait()


@functools.partial(jax.jit)
def _sc_gather(x_tb, slabs):
    mesh = plsc.VectorSubcoreMesh(
        core_axis_name="c", subcore_axis_name="s", num_cores=_NC, num_subcores=_NS
    )
    return pl.kernel(
        _gather_body,
        out_type=jax.ShapeDtypeStruct((_N, _D), jnp.float32),
        mesh=mesh,
        compiler_params=pltpu.CompilerParams(needs_layout_passes=False),
        scratch_types=(
            [pltpu.VMEM((_RS,), jnp.int32), pltpu.VMEM((_TPAD,), jnp.int32)]
            + [pltpu.VMEM((_C,), jnp.int32)] * 2
            + [pltpu.VMEM((_C, _D), jnp.float32)] * 2
            + [pltpu.SemaphoreType.DMA] * 4
        ),
    )(x_tb, slabs)


def kernel(inputs, neg_indices):
    x_tb = jnp.transpose(inputs, (1, 0, 2)).reshape(_T * _B, _D)
    l = jnp.arange(_TARGET_LEN, dtype=jnp.int32)
    p = jnp.arange(_PRED_STEPS, dtype=jnp.int32)
    pos = (_T_SKIP + 1 + l[:, None] + p[None, :]).reshape(-1)
    slabs = jnp.concatenate(
        [pos, neg_indices, jnp.zeros((_NW * _RS + _TPAD - _NSLAB,), jnp.int32)]
    )
    out = _sc_gather(x_tb, slabs)
    out = out.reshape(_NUM_NEG + 1, _TARGET_LEN, _PRED_STEPS, _B, _D)
    return jnp.transpose(out, (3, 0, 1, 2, 4))


# speedup vs baseline: 1.4826x; 1.0378x over previous
import jax
import jax.numpy as jnp
from jax import lax
from jax.experimental import pallas as pl
from jax.experimental.pallas import tpu as pltpu, tpu_sc as plsc
import functools

_T_SKIP = 4
_PRED_STEPS = 12
_NUM_NEG = 10
_B, _T, _D = 4, 512, 128
_TARGET_LEN = _T - _T_SKIP - _PRED_STEPS - 1
_NSLAB = (_NUM_NEG + 1) * _TARGET_LEN * _PRED_STEPS
_N = _NSLAB * _B

_NC, _NS = 2, 16
_NW = _NC * _NS
_RS = 2040
_R = _RS * _B
_TS = _NSLAB - _NW * _RS
_TPAD = 64
_C = 480
_CS = _C // _B
_NCHUNK = _R // _C
_NPAIR = (_NCHUNK - 1) // 2

_LANE = 16


def _build_idx(idx_slab, idx_c, nrows, row0):
    iota = lax.iota(jnp.int32, _LANE)
    for k in range(nrows // _LANE):
        r = row0 + k * _LANE + iota
        t = plsc.load_gather(idx_slab, [lax.shift_right_logical(r, 2)])
        idx_c[pl.ds(k * _LANE, _LANE)] = t * _B + lax.bitwise_and(r, 3)


def _gather_body(x_hbm, slab_hbm, out_hbm, idx_t, idx_tt, ic0, ic1,
                 buf0, buf1, g0, g1, s0, s1):
    ics = (ic0, ic1)
    bufs = (buf0, buf1)
    gsem = (g0, g1)
    ssem = (s0, s1)
    wid = lax.axis_index("s") * _NC + lax.axis_index("c")
    base = wid * _R
    pltpu.sync_copy(slab_hbm.at[pl.ds(wid * _RS, _RS)], idx_t)

    def drain_store(p):
        pltpu.make_async_copy(bufs[p], out_hbm.at[pl.ds(0, _C)], ssem[p]).wait()

    def wait_gather(p):
        pltpu.make_async_copy(x_hbm.at[ics[p]], bufs[p], gsem[p]).wait()

    def step(c, p, drain):
        if drain:
            drain_store(p)
        _build_idx(idx_t, ics[p], _C, c * _C)
        pltpu.async_copy(x_hbm.at[ics[p]], bufs[p], gsem[p])
        wait_gather(1 - p)
        pltpu.async_copy(
            bufs[1 - p], out_hbm.at[pl.ds(base + (c - 1) * _C, _C)], ssem[1 - p]
        )

    _build_idx(idx_t, ics[0], _C, 0)
    pltpu.async_copy(x_hbm.at[ics[0]], bufs[0], gsem[0])

    def pair(t, _):
        @pl.when(t > 0)
        def _():
            drain_store(1)

        step(2 * t + 1, 1, False)
        step(2 * t + 2, 0, True)
        return ()

    lax.fori_loop(0, _NPAIR, pair, (), unroll=False)
    wait_gather(0)
    pltpu.async_copy(
        bufs[0], out_hbm.at[pl.ds(base + (_NCHUNK - 1) * _C, _C)], ssem[0]
    )
    for p in range(2):
        drain_store(p)

    @pl.when(wid == _NW - 1)
    def _():
        nrows = _TS * _B
        pltpu.sync_copy(slab_hbm.at[pl.ds(_NW * _RS, _TPAD)], idx_tt)
        _build_idx(idx_tt, ic0, nrows, 0)
        pltpu.async_copy(
            x_hbm.at[ic0.at[pl.ds(0, nrows)]], buf0.at[pl.ds(0, nrows)], g0
        ).wait()
        pltpu.async_copy(
            buf0.at[pl.ds(0, nrows)], out_hbm.at[pl.ds(_NW * _R, nrows)], s0
        ).wait()


@functools.partial(jax.jit)
def _sc_gather(x_tb, slabs):
    mesh = plsc.VectorSubcoreMesh(
        core_axis_name="c", subcore_axis_name="s", num_cores=_NC, num_subcores=_NS
    )
    return pl.kernel(
        _gather_body,
        out_type=jax.ShapeDtypeStruct((_N, _D), jnp.float32),
        mesh=mesh,
        compiler_params=pltpu.CompilerParams(needs_layout_passes=False),
        scratch_types=(
            [pltpu.VMEM((_RS,), jnp.int32), pltpu.VMEM((_TPAD,), jnp.int32)]
            + [pltpu.VMEM((_C,), jnp.int32)] * 2
            + [pltpu.VMEM((_C, _D), jnp.float32)] * 2
            + [pltpu.SemaphoreType.DMA] * 4
        ),
    )(x_tb, slabs)


def kernel(inputs, neg_indices):
    x_tb = jnp.transpose(inputs, (1, 0, 2)).reshape(_T * _B, _D)
    l = jnp.arange(_TARGET_LEN, dtype=jnp.int32)
    p = jnp.arange(_PRED_STEPS, dtype=jnp.int32)
    pos = (_T_SKIP + 1 + l[:, None] + p[None, :]).reshape(-1)
    slabs = jnp.concatenate(
        [pos, neg_indices, jnp.zeros((_NW * _RS + _TPAD - _NSLAB,), jnp.int32)]
    )
    out = _sc_gather(x_tb, slabs)
    out = out.reshape(_NUM_NEG + 1, _TARGET_LEN, _PRED_STEPS, _B, _D)
    return jnp.transpose(out, (3, 0, 1, 2, 4))


# speedup vs baseline: 1.5136x; 1.0209x over previous
import jax
import jax.numpy as jnp
from jax import lax
from jax.experimental import pallas as pl
from jax.experimental.pallas import tpu as pltpu, tpu_sc as plsc
import functools

_T_SKIP = 4
_PRED_STEPS = 12
_NUM_NEG = 10
_B, _T, _D = 4, 512, 128
_TARGET_LEN = _T - _T_SKIP - _PRED_STEPS - 1
_NPOS = _TARGET_LEN * _PRED_STEPS
_NNEG = _NUM_NEG * _NPOS
_NSLAB = _NPOS + _NNEG
_N = _NSLAB * _B

_NC, _NS = 2, 16
_NW = _NC * _NS
_RS = 2040
_R = _RS * _B
_TS = _NSLAB - _NW * _RS
_WLEN = 2056
_TWLEN = 64
_TW0 = _NW * _RS - _NPOS - ((_NW * _RS - _NPOS) % 8)
_C = 480
_NCHUNK = _R // _C
_NPAIR = (_NCHUNK - 1) // 2

_LANE = 16


def _build_idx(negwin, win0, idx_c, nrows, slab0):
    iota = lax.iota(jnp.int32, _LANE)
    for k in range(nrows // _LANE):
        r = (slab0 * _B) + k * _LANE + iota
        sg = lax.shift_right_logical(r, 2)
        t_pos = (_T_SKIP + 1) + sg // _PRED_STEPS + sg % _PRED_STEPS
        w = jnp.clip(sg - _NPOS - win0, 0, _WLEN - 1)
        t_neg = plsc.load_gather(negwin, [w])
        t = jnp.where(sg < _NPOS, t_pos, t_neg)
        idx_c[pl.ds(k * _LANE, _LANE)] = t * _B + lax.bitwise_and(r, 3)


def _gather_body(x_hbm, neg_hbm, out_hbm, negwin, ic0, ic1,
                 buf0, buf1, g0, g1, s0, s1):
    ics = (ic0, ic1)
    bufs = (buf0, buf1)
    gsem = (g0, g1)
    ssem = (s0, s1)
    wid = lax.axis_index("s") * _NC + lax.axis_index("c")
    base = wid * _R
    ws = wid * _RS
    win0 = jnp.maximum(ws - _NPOS, 0) & ~7
    win0 = pl.multiple_of(jnp.minimum(win0, _NNEG - _WLEN), 8)
    pltpu.sync_copy(neg_hbm.at[pl.ds(win0, _WLEN)], negwin)

    def drain_store(p):
        pltpu.make_async_copy(bufs[p], out_hbm.at[pl.ds(0, _C)], ssem[p]).wait()

    def wait_gather(p):
        pltpu.make_async_copy(x_hbm.at[ics[p]], bufs[p], gsem[p]).wait()

    def step(c, p, drain):
        if drain:
            drain_store(p)
        _build_idx(negwin, win0, ics[p], _C, ws + c * (_C // _B))
        pltpu.async_copy(x_hbm.at[ics[p]], bufs[p], gsem[p])
        wait_gather(1 - p)
        pltpu.async_copy(
            bufs[1 - p], out_hbm.at[pl.ds(base + (c - 1) * _C, _C)], ssem[1 - p]
        )

    _build_idx(negwin, win0, ics[0], _C, ws)
    pltpu.async_copy(x_hbm.at[ics[0]], bufs[0], gsem[0])

    def pair(t, _):
        @pl.when(t > 0)
        def _():
            drain_store(1)

        step(2 * t + 1, 1, False)
        step(2 * t + 2, 0, True)
        return ()

    lax.fori_loop(0, _NPAIR, pair, (), unroll=False)
    wait_gather(0)
    pltpu.async_copy(
        bufs[0], out_hbm.at[pl.ds(base + (_NCHUNK - 1) * _C, _C)], ssem[0]
    )
    for p in range(2):
        drain_store(p)

    @pl.when(wid == _NW - 1)
    def _():
        nrows = _TS * _B
        pltpu.sync_copy(neg_hbm.at[pl.ds(_TW0, _TWLEN)], negwin.at[pl.ds(0, _TWLEN)])
        _build_idx(negwin, jnp.int32(_TW0), ic0, nrows, _NW * _RS)
        pltpu.async_copy(
            x_hbm.at[ic0.at[pl.ds(0, nrows)]], buf0.at[pl.ds(0, nrows)], g0
        ).wait()
        pltpu.async_copy(
            buf0.at[pl.ds(0, nrows)], out_hbm.at[pl.ds(_NW * _R, nrows)], s0
        ).wait()


@functools.partial(jax.jit)
def _sc_gather(x_tb, neg_indices):
    mesh = plsc.VectorSubcoreMesh(
        core_axis_name="c", subcore_axis_name="s", num_cores=_NC, num_subcores=_NS
    )
    return pl.kernel(
        _gather_body,
        out_type=jax.ShapeDtypeStruct((_N, _D), jnp.float32),
        mesh=mesh,
        compiler_params=pltpu.CompilerParams(needs_layout_passes=False),
        scratch_types=(
            [pltpu.VMEM((_WLEN,), jnp.int32)]
            + [pltpu.VMEM((_C,), jnp.int32)] * 2
            + [pltpu.VMEM((_C, _D), jnp.float32)] * 2
            + [pltpu.SemaphoreType.DMA] * 4
        ),
    )(x_tb, neg_indices)


def kernel(inputs, neg_indices):
    x_tb = jnp.transpose(inputs, (1, 0, 2)).reshape(_T * _B, _D)
    out = _sc_gather(x_tb, neg_indices)
    out = out.reshape(_NUM_NEG + 1, _TARGET_LEN, _PRED_STEPS, _B, _D)
    return jnp.transpose(out, (3, 0, 1, 2, 4))
